# Initial kernel scaffold; baseline (speedup 1.0000x reference)
import jax
import jax.numpy as jnp
from jax.experimental import pallas as pl


def kernel(x_seq, edge_index, edge_weight, Wz, bz, Wr, br, Wh, bh, Wlz, blz, Wlr, blr, Wlh, blh, Wout, bout):
    raise NotImplementedError("write your pallas kernel here")


# speedup vs baseline: 4.2441x; 4.2441x over previous
import functools

import jax
import jax.numpy as jnp
from jax import lax
from jax.experimental import pallas as pl


def _dense_body(y_ref, Wz_ref, Wr_ref, Wh_ref, bz_ref, br_ref, bh_ref,
                Wlz_ref, Wlr_ref, Wlh_ref, blz_ref, blr_ref, blh_ref,
                Wout_ref, bout_ref, out_ref, *, T, NB, F, HS):
    f32 = jnp.float32
    dot = functools.partial(jnp.dot, preferred_element_type=f32)
    Wlz = Wlz_ref[...]
    Wlr = Wlr_ref[...]
    Wlh = Wlh_ref[...]
    Cz = dot(Wz_ref[...], Wlz[:HS])
    Cr = dot(Wr_ref[...], Wlr[:HS])
    Ch = dot(Wh_ref[...], Wlh[:HS])
    cz = dot(bz_ref[...], Wlz[:HS]) + blz_ref[...]
    cr = dot(br_ref[...], Wlr[:HS]) + blr_ref[...]
    ch = dot(bh_ref[...], Wlh[:HS]) + blh_ref[...]
    Uz, Ur, Uh = Wlz[HS:], Wlr[HS:], Wlh[HS:]

    y = y_ref[...]
    Ccat = jnp.concatenate([Cz, Cr, Ch], axis=1)
    P = dot(y.reshape(T * NB, F), Ccat).reshape(T, NB, 3 * HS)

    h = jnp.zeros((NB, HS), f32)
    for t in range(T):
        Z = jax.nn.sigmoid(P[t, :, :HS] + dot(h, Uz) + cz)
        R = jax.nn.sigmoid(P[t, :, HS:2 * HS] + dot(h, Ur) + cr)
        Ht = jnp.tanh(P[t, :, 2 * HS:] + dot(h * R, Uh) + ch)
        h = Z * h + (1.0 - Z) * Ht
    out_ref[...] = dot(h, Wout_ref[...]) + bout_ref[...]


def _dense_stage(y_all, Wz, Wr, Wh, bz, br, bh, Wlz, Wlr, Wlh,
                 blz, blr, blh, Wout, bout):
    T, N, F = y_all.shape
    HS = Wz.shape[1]
    NB = 2000 if N % 2000 == 0 else N
    grid = (N // NB,)
    full = lambda a: pl.BlockSpec(a.shape, lambda i: (0,) * a.ndim)
    return pl.pallas_call(
        functools.partial(_dense_body, T=T, NB=NB, F=F, HS=HS),
        grid=grid,
        in_specs=[
            pl.BlockSpec((T, NB, F), lambda i: (0, i, 0)),
            full(Wz), full(Wr), full(Wh),
            full(bz), full(br), full(bh),
            full(Wlz), full(Wlr), full(Wlh),
            full(blz), full(blr), full(blh),
            full(Wout), full(bout),
        ],
        out_specs=pl.BlockSpec((NB, 1), lambda i: (i, 0)),
        out_shape=jax.ShapeDtypeStruct((N, 1), jnp.float32),
    )(y_all, Wz, Wr, Wh, bz, br, bh, Wlz, Wlr, Wlh, blz, blr, blh,
      Wout, bout)


def kernel(x_seq, edge_index, edge_weight, Wz, bz, Wr, br, Wh, bh,
           Wlz, blz, Wlr, blr, Wlh, blh, Wout, bout):
    N, T, F = x_seq.shape
    HS = Wz.shape[1]
    row, col = edge_index[0], edge_index[1]

    deg = jnp.zeros((N,), jnp.float32).at[col].add(edge_weight) + 1.0
    dinv = lax.rsqrt(deg)
    Xp = x_seq * dinv[:, None, None]
    ys = []
    for t in range(T):
        xt = Xp[:, t, :]
        msg = jnp.take(xt, row, axis=0) * edge_weight[:, None]
        S = jnp.zeros((N, F), jnp.float32).at[col].add(msg)
        ys.append(dinv[:, None] * (S + xt))
    y_all = jnp.stack(ys, axis=0)

    return _dense_stage(
        y_all, Wz, Wr, Wh,
        bz.reshape(1, HS), br.reshape(1, HS), bh.reshape(1, HS),
        Wlz, Wlr, Wlh,
        blz.reshape(1, HS), blr.reshape(1, HS), blh.reshape(1, HS),
        Wout, bout.reshape(1, 1))


# speedup vs baseline: 14.1839x; 3.3420x over previous
import functools

import jax
import jax.numpy as jnp
from jax import lax
from jax.experimental import pallas as pl
from jax.experimental.pallas import tpu as pltpu
from jax.experimental.pallas import tpu_sc as plsc

_N = 10000
_T = 8
_F = 128
_NC = 2
_NS = 16
_NW = _NC * _NS
_K = 80
_RCH = 80
_NCHK = _N // _RCH
_CPS = -(-_NCHK // _NS)


def _full16(v):
    return jnp.full((16,), v, jnp.int32)


def _node_chunks(s):
    first = s * _CPS
    num = jnp.maximum(0, jnp.minimum(_CPS, _NCHK - first))
    return first, num


def _deg_body(nchunk_pw, eidx, ewc, zeros, out, dacc, ewb, colb, vbuf, _sem):
    c = lax.axis_index("c")
    s = lax.axis_index("s")
    wid = s * _NC + c
    cbase = wid * nchunk_pw
    first, num = _node_chunks(s)

    @pl.loop(0, num)
    def _zero(k):
        pltpu.sync_copy(zeros, dacc.at[pl.ds((first + k) * _RCH, _RCH)])

    plsc.subcore_barrier()

    @pl.loop(0, nchunk_pw)
    def _chunk(i):
        pltpu.sync_copy(eidx.at[pl.ds((cbase + i) * (2 * _K) + _K, _K)], colb)
        pltpu.sync_copy(ewc.at[pl.ds((cbase + i) * _K, _K)], ewb.at[pl.ds(16, _K)])
        for e in range(_K):
            spl = plsc.load_gather(ewb, [_full16(16 + e)])
            for j in range(_F // 16):
                vbuf[e, pl.ds(j * 16, 16)] = spl
        pltpu.sync_copy(vbuf, dacc.at[colb], add=True)

    plsc.subcore_barrier()

    @pl.loop(0, num)
    def _dump(k):
        r0 = (first + k) * _RCH
        pltpu.sync_copy(dacc.at[pl.ds(r0, _RCH)], vbuf)
        pltpu.sync_copy(vbuf, out.at[c, pl.ds(r0, _RCH)])


def _deg_sc(eidx, ewc, nchunk_pw):
    mesh = plsc.VectorSubcoreMesh(core_axis_name="c", subcore_axis_name="s")
    zeros = jnp.zeros((_RCH, _F), jnp.float32)
    return pl.kernel(
        functools.partial(_deg_body, nchunk_pw),
        out_type=jax.ShapeDtypeStruct((_NC, _N, _F), jnp.float32),
        mesh=mesh,
        compiler_params=pltpu.CompilerParams(needs_layout_passes=False),
        scratch_types=[
            pltpu.VMEM_SHARED((_N, _F), jnp.float32),
            pltpu.VMEM((16 + _K,), jnp.float32),
            pltpu.VMEM((_K,), jnp.int32),
            pltpu.VMEM((_K, _F), jnp.float32),
            pltpu.SemaphoreType.DMA,
        ],
    )(eidx, ewc, zeros)


def _agg_body(nchunk_pw, eidx, ewc, xp, zeros, out, accum, edb, ewb, idxb, colb,
              gbuf, dbuf, sem):
    c = lax.axis_index("c")
    s = lax.axis_index("s")
    wid = s * _NC + c
    cbase = wid * nchunk_pw
    first, num = _node_chunks(s)

    @pl.loop(0, num)
    def _zero(k):
        pltpu.sync_copy(zeros, accum.at[pl.ds((first + k) * _RCH, _RCH)])

    plsc.subcore_barrier()

    @pl.loop(0, _T)
    def _t_loop(t):
        toff = t * _N

        @pl.loop(0, nchunk_pw)
        def _chunk(i):
            off = (cbase + i) * (2 * _K)
            pltpu.sync_copy(eidx.at[pl.ds(off, 2 * _K)], edb)
            pltpu.sync_copy(ewc.at[pl.ds((cbase + i) * _K, _K)],
                            ewb.at[pl.ds(16, _K)])
            for g in range(_K // 16):
                sl = pl.ds(g * 16, 16)
                idxb[sl] = edb[sl] + toff
                colb[sl] = edb[pl.ds(_K + g * 16, 16)]
            pltpu.async_copy(xp.at[idxb], gbuf, sem).wait()
            for e in range(_K):
                spl = plsc.load_gather(ewb, [_full16(16 + e)])
                for j in range(_F // 16):
                    cs = pl.ds(j * 16, 16)
                    gbuf[e, cs] = gbuf[e, cs] * spl
            pltpu.sync_copy(gbuf, accum.at[colb], add=True)

        plsc.subcore_barrier()

        @pl.loop(0, num)
        def _dump(k):
            r0 = (first + k) * _RCH
            pltpu.sync_copy(accum.at[pl.ds(r0, _RCH)], dbuf)
            pltpu.sync_copy(dbuf, out.at[c, t, pl.ds(r0, _RCH)])

        plsc.subcore_barrier()


def _agg_sc(eidx, ewc, xp_flat, nchunk_pw):
    mesh = plsc.VectorSubcoreMesh(core_axis_name="c", subcore_axis_name="s")
    zeros = jnp.zeros((_RCH, _F), jnp.float32)
    return pl.kernel(
        functools.partial(_agg_body, nchunk_pw),
        out_type=jax.ShapeDtypeStruct((_NC, _T, _N, _F), jnp.float32),
        mesh=mesh,
        compiler_params=pltpu.CompilerParams(needs_layout_passes=False),
        scratch_types=[
            pltpu.VMEM_SHARED((_N, _F), jnp.float32),
            pltpu.VMEM((2 * _K,), jnp.int32),
            pltpu.VMEM((16 + _K,), jnp.float32),
            pltpu.VMEM((_K,), jnp.int32),
            pltpu.VMEM((_K,), jnp.int32),
            pltpu.VMEM((_K, _F), jnp.float32),
            pltpu.VMEM((_RCH, _F), jnp.float32),
            pltpu.SemaphoreType.DMA,
        ],
    )(eidx, ewc, xp_flat, zeros)


def _prep_body(x_ref, d_ref, xp_ref, dinv_ref, *, T):
    dsum = 1.0 + d_ref[0, :, 0] + d_ref[1, :, 0]
    dinv = lax.rsqrt(dsum)
    dinv_ref[...] = dinv[:, None]
    x = x_ref[...]
    xp_ref[...] = jnp.transpose(x, (1, 0, 2)) * dinv[None, :, None]


def _prep_tc(x_seq, degdump):
    N, T, F = x_seq.shape
    NB = 2000
    grid = (N // NB,)
    return pl.pallas_call(
        functools.partial(_prep_body, T=T),
        grid=grid,
        in_specs=[
            pl.BlockSpec((NB, T, F), lambda i: (i, 0, 0)),
            pl.BlockSpec((2, NB, _F), lambda i: (0, i, 0)),
        ],
        out_specs=[
            pl.BlockSpec((T, NB, F), lambda i: (0, i, 0)),
            pl.BlockSpec((NB, 1), lambda i: (i, 0)),
        ],
        out_shape=[
            jax.ShapeDtypeStruct((T, N, F), jnp.float32),
            jax.ShapeDtypeStruct((N, 1), jnp.float32),
        ],
    )(x_seq, degdump)


def _dense_body(S_ref, xp_ref, dinv_ref, Wz_ref, Wr_ref, Wh_ref,
                bz_ref, br_ref, bh_ref, Wlz_ref, Wlr_ref, Wlh_ref,
                blz_ref, blr_ref, blh_ref, Wout_ref, bout_ref, out_ref,
                *, T, NB, F, HS):
    f32 = jnp.float32
    dot = functools.partial(jnp.dot, preferred_element_type=f32)
    Wlz = Wlz_ref[...]
    Wlr = Wlr_ref[...]
    Wlh = Wlh_ref[...]
    Cz = dot(Wz_ref[...], Wlz[:HS])
    Cr = dot(Wr_ref[...], Wlr[:HS])
    Ch = dot(Wh_ref[...], Wlh[:HS])
    cz = dot(bz_ref[...], Wlz[:HS]) + blz_ref[...]
    cr = dot(br_ref[...], Wlr[:HS]) + blr_ref[...]
    ch = dot(bh_ref[...], Wlh[:HS]) + blh_ref[...]
    Uz, Ur, Uh = Wlz[HS:], Wlr[HS:], Wlh[HS:]

    dinv = dinv_ref[...]
    S = S_ref[...]
    xp = xp_ref[...]
    Ssum = S[0] + S[1]
    ys = [Ssum[0] + xp[0]]
    for t in range(1, T):
        ys.append(Ssum[t] - Ssum[t - 1] + xp[t])
    y = jnp.stack(ys, axis=0) * dinv[None, :, :]

    Ccat = jnp.concatenate([Cz, Cr, Ch], axis=1)
    P = dot(y.reshape(T * NB, F), Ccat).reshape(T, NB, 3 * HS)

    h = jnp.zeros((NB, HS), f32)
    for t in range(T):
        Z = jax.nn.sigmoid(P[t, :, :HS] + dot(h, Uz) + cz)
        R = jax.nn.sigmoid(P[t, :, HS:2 * HS] + dot(h, Ur) + cr)
        Ht = jnp.tanh(P[t, :, 2 * HS:] + dot(h * R, Uh) + ch)
        h = Z * h + (1.0 - Z) * Ht
    out_ref[...] = dot(h, Wout_ref[...]) + bout_ref[...]


def _dense_tc(S_cum, xp, dinv, Wz, Wr, Wh, bz, br, bh, Wlz, Wlr, Wlh,
              blz, blr, blh, Wout, bout):
    T, N, F = xp.shape
    HS = Wz.shape[1]
    NB = 1000
    grid = (N // NB,)
    full = lambda a: pl.BlockSpec(a.shape, lambda i: (0,) * a.ndim)
    return pl.pallas_call(
        functools.partial(_dense_body, T=T, NB=NB, F=F, HS=HS),
        grid=grid,
        in_specs=[
            pl.BlockSpec((2, T, NB, F), lambda i: (0, 0, i, 0)),
            pl.BlockSpec((T, NB, F), lambda i: (0, i, 0)),
            pl.BlockSpec((NB, 1), lambda i: (i, 0)),
            full(Wz), full(Wr), full(Wh),
            full(bz), full(br), full(bh),
            full(Wlz), full(Wlr), full(Wlh),
            full(blz), full(blr), full(blh),
            full(Wout), full(bout),
        ],
        out_specs=pl.BlockSpec((NB, 1), lambda i: (i, 0)),
        out_shape=jax.ShapeDtypeStruct((N, 1), jnp.float32),
    )(S_cum, xp, dinv, Wz, Wr, Wh, bz, br, bh, Wlz, Wlr, Wlh,
      blz, blr, blh, Wout, bout)


def kernel(x_seq, edge_index, edge_weight, Wz, bz, Wr, br, Wh, bh,
           Wlz, blz, Wlr, blr, Wlh, blh, Wout, bout):
    N, T, F = x_seq.shape
    HS = Wz.shape[1]
    E = edge_index.shape[1]
    nchunks = E // _K
    nchunk_pw = nchunks // _NW

    eidx = (jnp.stack([edge_index[0], edge_index[1]], axis=0)
            .reshape(2, nchunks, _K).transpose(1, 0, 2).reshape(-1))
    ewc = edge_weight

    degdump = _deg_sc(eidx, ewc, nchunk_pw)
    xp, dinv = _prep_tc(x_seq, degdump)
    S_cum = _agg_sc(eidx, ewc, xp.reshape(T * N, F), nchunk_pw)

    return _dense_tc(
        S_cum, xp, dinv, Wz, Wr, Wh,
        bz.reshape(1, HS), br.reshape(1, HS), bh.reshape(1, HS),
        Wlz, Wlr, Wlh,
        blz.reshape(1, HS), blr.reshape(1, HS), blh.reshape(1, HS),
        Wout, bout.reshape(1, 1))


# speedup vs baseline: 19.3252x; 1.3625x over previous
import functools

import jax
import jax.numpy as jnp
from jax import lax
from jax.experimental import pallas as pl
from jax.experimental.pallas import tpu as pltpu
from jax.experimental.pallas import tpu_sc as plsc

_N = 10000
_T = 8
_F = 128
_NC = 2
_NS = 16
_NW = _NC * _NS
_K = 80
_RCH = 80
_NCHK = _N // _RCH
_CPS = -(-_NCHK // _NS)


def _full16(v):
    return jnp.full((16,), v, jnp.int32)


def _node_chunks(s):
    first = s * _CPS
    num = jnp.maximum(0, jnp.minimum(_CPS, _NCHK - first))
    return first, num


def _deg_body(nchunk_pw, eidx, ewc, zeros, out, dacc, ewb, colb, vbuf, _sem):
    c = lax.axis_index("c")
    s = lax.axis_index("s")
    wid = s * _NC + c
    cbase = wid * nchunk_pw
    first, num = _node_chunks(s)

    @pl.loop(0, num)
    def _zero(k):
        pltpu.sync_copy(zeros, dacc.at[pl.ds((first + k) * _RCH, _RCH)])

    plsc.subcore_barrier()

    @pl.loop(0, nchunk_pw)
    def _chunk(i):
        pltpu.sync_copy(eidx.at[pl.ds((cbase + i) * (2 * _K) + _K, _K)], colb)
        pltpu.sync_copy(ewc.at[pl.ds((cbase + i) * _K, _K)], ewb.at[pl.ds(16, _K)])
        for e in range(_K):
            spl = plsc.load_gather(ewb, [_full16(16 + e)])
            for j in range(_F // 16):
                vbuf[e, pl.ds(j * 16, 16)] = spl
        pltpu.sync_copy(vbuf, dacc.at[colb], add=True)

    plsc.subcore_barrier()

    @pl.loop(0, num)
    def _dump(k):
        r0 = (first + k) * _RCH
        pltpu.sync_copy(dacc.at[pl.ds(r0, _RCH)], vbuf)
        pltpu.sync_copy(vbuf, out.at[c, pl.ds(r0, _RCH)])


def _deg_sc(eidx, ewc, nchunk_pw):
    mesh = plsc.VectorSubcoreMesh(core_axis_name="c", subcore_axis_name="s")
    zeros = jnp.zeros((_RCH, _F), jnp.float32)
    return pl.kernel(
        functools.partial(_deg_body, nchunk_pw),
        out_type=jax.ShapeDtypeStruct((_NC, _N, _F), jnp.float32),
        mesh=mesh,
        compiler_params=pltpu.CompilerParams(needs_layout_passes=False),
        scratch_types=[
            pltpu.VMEM_SHARED((_N, _F), jnp.float32),
            pltpu.VMEM((16 + _K,), jnp.float32),
            pltpu.VMEM((_K,), jnp.int32),
            pltpu.VMEM((_K, _F), jnp.float32),
            pltpu.SemaphoreType.DMA,
        ],
    )(eidx, ewc, zeros)


def _agg_body(nchunk_pw, eidx, ewc, xp, zeros, out, accum, edball,
              ewb0, ewb1, idxb0, idxb1, colb0, colb1, gbuf0, gbuf1,
              sg0, sg1, ss0, ss1):
    c = lax.axis_index("c")
    s = lax.axis_index("s")
    wid = s * _NC + c
    cbase = wid * nchunk_pw
    first, num = _node_chunks(s)
    ewb = (ewb0, ewb1)
    idxb = (idxb0, idxb1)
    colb = (colb0, colb1)
    gbuf = (gbuf0, gbuf1)
    sg = (sg0, sg1)
    ss = (ss0, ss1)

    pltpu.sync_copy(eidx.at[pl.ds(cbase * (2 * _K), nchunk_pw * 2 * _K)], edball)

    @pl.loop(0, num)
    def _zero(k):
        pltpu.sync_copy(zeros, accum.at[pl.ds((first + k) * _RCH, _RCH)])

    plsc.subcore_barrier()

    def issue(q, p, toff):
        for g in range(_K // 16):
            sl = pl.ds(g * 16, 16)
            idxb[p][sl] = edball[pl.ds(q * (2 * _K) + g * 16, 16)] + toff
            colb[p][sl] = edball[pl.ds(q * (2 * _K) + _K + g * 16, 16)]
        pltpu.async_copy(ewc.at[pl.ds((cbase + q) * _K, _K)],
                         ewb[p].at[pl.ds(16, _K)], sg[p])
        pltpu.async_copy(xp.at[idxb[p]], gbuf[p], sg[p])

    def process(q, p):
        pltpu.make_async_copy(ewc.at[pl.ds((cbase + q) * _K, _K)],
                              ewb[p].at[pl.ds(16, _K)], sg[p]).wait()
        pltpu.make_async_copy(xp.at[idxb[p]], gbuf[p], sg[p]).wait()
        for e in range(_K):
            spl = plsc.load_gather(ewb[p], [_full16(16 + e)])
            for j in range(_F // 16):
                cs = pl.ds(j * 16, 16)
                gbuf[p][e, cs] = gbuf[p][e, cs] * spl
        pltpu.async_copy(gbuf[p], accum.at[colb[p]], ss[p], add=True)

    def drain(p):
        pltpu.make_async_copy(gbuf[p], accum.at[colb[p]], ss[p]).wait()

    npairs = (nchunk_pw - 1) // 2

    @pl.loop(0, _T)
    def _t_loop(t):
        toff = t * _N
        issue(0, 0, toff)

        @pl.loop(0, npairs)
        def _pair(j):
            i = 2 * j

            @pl.when(j > 0)
            def _():
                drain(1)

            issue(i + 1, 1, toff)
            process(i, 0)
            drain(0)
            issue(i + 2, 0, toff)
            process(i + 1, 1)

        process(nchunk_pw - 1, 0)
        drain(0)
        drain(1)
        plsc.subcore_barrier()

        @pl.loop(0, num)
        def _dump(k):
            r0 = (first + k) * _RCH
            pltpu.sync_copy(accum.at[pl.ds(r0, _RCH)], gbuf0)
            pltpu.sync_copy(gbuf0, out.at[c, t, pl.ds(r0, _RCH)])

        plsc.subcore_barrier()


def _agg_sc(eidx, ewc, xp_flat, nchunk_pw):
    mesh = plsc.VectorSubcoreMesh(core_axis_name="c", subcore_axis_name="s")
    zeros = jnp.zeros((_RCH, _F), jnp.float32)
    ne_pw = nchunk_pw * _K
    return pl.kernel(
        functools.partial(_agg_body, nchunk_pw),
        out_type=jax.ShapeDtypeStruct((_NC, _T, _N, _F), jnp.float32),
        mesh=mesh,
        compiler_params=pltpu.CompilerParams(needs_layout_passes=False),
        scratch_types=[
            pltpu.VMEM_SHARED((_N, _F), jnp.float32),
            pltpu.VMEM((2 * ne_pw,), jnp.int32),
            pltpu.VMEM((16 + _K,), jnp.float32),
            pltpu.VMEM((16 + _K,), jnp.float32),
            pltpu.VMEM((_K,), jnp.int32),
            pltpu.VMEM((_K,), jnp.int32),
            pltpu.VMEM((_K,), jnp.int32),
            pltpu.VMEM((_K,), jnp.int32),
            pltpu.VMEM((_K, _F), jnp.float32),
            pltpu.VMEM((_K, _F), jnp.float32),
            pltpu.SemaphoreType.DMA,
            pltpu.SemaphoreType.DMA,
            pltpu.SemaphoreType.DMA,
            pltpu.SemaphoreType.DMA,
        ],
    )(eidx, ewc, xp_flat, zeros)


def _prep_body(x_ref, d_ref, xp_ref, dinv_ref, *, T):
    dsum = 1.0 + d_ref[0, :, 0] + d_ref[1, :, 0]
    dinv = lax.rsqrt(dsum)
    dinv_ref[...] = dinv[:, None]
    x = x_ref[...]
    xp_ref[...] = jnp.transpose(x, (1, 0, 2)) * dinv[None, :, None]


def _prep_tc(x_seq, degdump):
    N, T, F = x_seq.shape
    NB = 2000
    grid = (N // NB,)
    return pl.pallas_call(
        functools.partial(_prep_body, T=T),
        grid=grid,
        in_specs=[
            pl.BlockSpec((NB, T, F), lambda i: (i, 0, 0)),
            pl.BlockSpec((2, NB, _F), lambda i: (0, i, 0)),
        ],
        out_specs=[
            pl.BlockSpec((T, NB, F), lambda i: (0, i, 0)),
            pl.BlockSpec((NB, 1), lambda i: (i, 0)),
        ],
        out_shape=[
            jax.ShapeDtypeStruct((T, N, F), jnp.float32),
            jax.ShapeDtypeStruct((N, 1), jnp.float32),
        ],
    )(x_seq, degdump)


def _dense_body(S_ref, xp_ref, dinv_ref, Wz_ref, Wr_ref, Wh_ref,
                bz_ref, br_ref, bh_ref, Wlz_ref, Wlr_ref, Wlh_ref,
                blz_ref, blr_ref, blh_ref, Wout_ref, bout_ref, out_ref,
                *, T, NB, F, HS):
    f32 = jnp.float32
    dot = functools.partial(jnp.dot, preferred_element_type=f32)
    Wlz = Wlz_ref[...]
    Wlr = Wlr_ref[...]
    Wlh = Wlh_ref[...]
    Cz = dot(Wz_ref[...], Wlz[:HS])
    Cr = dot(Wr_ref[...], Wlr[:HS])
    Ch = dot(Wh_ref[...], Wlh[:HS])
    cz = dot(bz_ref[...], Wlz[:HS]) + blz_ref[...]
    cr = dot(br_ref[...], Wlr[:HS]) + blr_ref[...]
    ch = dot(bh_ref[...], Wlh[:HS]) + blh_ref[...]
    Uz, Ur, Uh = Wlz[HS:], Wlr[HS:], Wlh[HS:]

    dinv = dinv_ref[...]
    S = S_ref[...]
    xp = xp_ref[...]
    Ssum = S[0] + S[1]
    ys = [Ssum[0] + xp[0]]
    for t in range(1, T):
        ys.append(Ssum[t] - Ssum[t - 1] + xp[t])
    y = jnp.stack(ys, axis=0) * dinv[None, :, :]

    Ccat = jnp.concatenate([Cz, Cr, Ch], axis=1)
    P = dot(y.reshape(T * NB, F), Ccat).reshape(T, NB, 3 * HS)

    h = jnp.zeros((NB, HS), f32)
    for t in range(T):
        Z = jax.nn.sigmoid(P[t, :, :HS] + dot(h, Uz) + cz)
        R = jax.nn.sigmoid(P[t, :, HS:2 * HS] + dot(h, Ur) + cr)
        Ht = jnp.tanh(P[t, :, 2 * HS:] + dot(h * R, Uh) + ch)
        h = Z * h + (1.0 - Z) * Ht
    out_ref[...] = dot(h, Wout_ref[...]) + bout_ref[...]


def _dense_tc(S_cum, xp, dinv, Wz, Wr, Wh, bz, br, bh, Wlz, Wlr, Wlh,
              blz, blr, blh, Wout, bout):
    T, N, F = xp.shape
    HS = Wz.shape[1]
    NB = 1000
    grid = (N // NB,)
    full = lambda a: pl.BlockSpec(a.shape, lambda i: (0,) * a.ndim)
    return pl.pallas_call(
        functools.partial(_dense_body, T=T, NB=NB, F=F, HS=HS),
        grid=grid,
        in_specs=[
            pl.BlockSpec((2, T, NB, F), lambda i: (0, 0, i, 0)),
            pl.BlockSpec((T, NB, F), lambda i: (0, i, 0)),
            pl.BlockSpec((NB, 1), lambda i: (i, 0)),
            full(Wz), full(Wr), full(Wh),
            full(bz), full(br), full(bh),
            full(Wlz), full(Wlr), full(Wlh),
            full(blz), full(blr), full(blh),
            full(Wout), full(bout),
        ],
        out_specs=pl.BlockSpec((NB, 1), lambda i: (i, 0)),
        out_shape=jax.ShapeDtypeStruct((N, 1), jnp.float32),
    )(S_cum, xp, dinv, Wz, Wr, Wh, bz, br, bh, Wlz, Wlr, Wlh,
      blz, blr, blh, Wout, bout)


def kernel(x_seq, edge_index, edge_weight, Wz, bz, Wr, br, Wh, bh,
           Wlz, blz, Wlr, blr, Wlh, blh, Wout, bout):
    N, T, F = x_seq.shape
    HS = Wz.shape[1]
    E = edge_index.shape[1]
    nchunks = E // _K
    nchunk_pw = nchunks // _NW

    eidx = (jnp.stack([edge_index[0], edge_index[1]], axis=0)
            .reshape(2, nchunks, _K).transpose(1, 0, 2).reshape(-1))
    ewc = edge_weight

    degdump = _deg_sc(eidx, ewc, nchunk_pw)
    xp, dinv = _prep_tc(x_seq, degdump)
    S_cum = _agg_sc(eidx, ewc, xp.reshape(T * N, F), nchunk_pw)

    return _dense_tc(
        S_cum, xp, dinv, Wz, Wr, Wh,
        bz.reshape(1, HS), br.reshape(1, HS), bh.reshape(1, HS),
        Wlz, Wlr, Wlh,
        blz.reshape(1, HS), blr.reshape(1, HS), blh.reshape(1, HS),
        Wout, bout.reshape(1, 1))


# speedup vs baseline: 20.1689x; 1.0437x over previous
import functools

import jax
import jax.numpy as jnp
from jax import lax
from jax.experimental import pallas as pl
from jax.experimental.pallas import tpu as pltpu
from jax.experimental.pallas import tpu_sc as plsc

_N = 10000
_T = 8
_F = 128
_NC = 2
_NS = 16
_NW = _NC * _NS
_K = 80
_RCH = 80
_NCHK = _N // _RCH
_CPS = -(-_NCHK // _NS)


def _full16(v):
    return jnp.full((16,), v, jnp.int32)


def _node_chunks(s):
    first = s * _CPS
    num = jnp.maximum(0, jnp.minimum(_CPS, _NCHK - first))
    return first, num


def _deg_body(nchunk_pw, cols_hbm, ewc, zeros, out, dacc, ewb, colb, vbuf, _sem):
    c = lax.axis_index("c")
    s = lax.axis_index("s")
    wid = s * _NC + c
    cbase = wid * nchunk_pw
    first, num = _node_chunks(s)

    @pl.loop(0, num)
    def _zero(k):
        pltpu.sync_copy(zeros, dacc.at[pl.ds((first + k) * _RCH, _RCH)])

    plsc.subcore_barrier()

    @pl.loop(0, nchunk_pw)
    def _chunk(i):
        pltpu.sync_copy(cols_hbm.at[pl.ds((cbase + i) * _K, _K)], colb)
        pltpu.sync_copy(ewc.at[pl.ds((cbase + i) * _K, _K)], ewb.at[pl.ds(16, _K)])
        for e in range(_K):
            spl = plsc.load_gather(ewb, [_full16(16 + e)])
            for j in range(_F // 16):
                vbuf[e, pl.ds(j * 16, 16)] = spl
        pltpu.sync_copy(vbuf, dacc.at[colb], add=True)

    plsc.subcore_barrier()

    @pl.loop(0, num)
    def _dump(k):
        r0 = (first + k) * _RCH
        pltpu.sync_copy(dacc.at[pl.ds(r0, _RCH)], vbuf)
        pltpu.sync_copy(vbuf, out.at[c, pl.ds(r0, _RCH)])


def _deg_sc(cols, ewc, nchunk_pw):
    mesh = plsc.VectorSubcoreMesh(core_axis_name="c", subcore_axis_name="s")
    zeros = jnp.zeros((_RCH, _F), jnp.float32)
    return pl.kernel(
        functools.partial(_deg_body, nchunk_pw),
        out_type=jax.ShapeDtypeStruct((_NC, _N, _F), jnp.float32),
        mesh=mesh,
        compiler_params=pltpu.CompilerParams(needs_layout_passes=False),
        scratch_types=[
            pltpu.VMEM_SHARED((_N, _F), jnp.float32),
            pltpu.VMEM((16 + _K,), jnp.float32),
            pltpu.VMEM((_K,), jnp.int32),
            pltpu.VMEM((_K, _F), jnp.float32),
            pltpu.SemaphoreType.DMA,
        ],
    )(cols, ewc, zeros)


def _agg_body(nchunk_pw, rows_hbm, cols_hbm, ewc, xp, zeros, out, accum,
              rowball, ewb0, ewb1, ewb2, idxb0, idxb1, idxb2,
              colb0, colb1, colb2, gbuf0, gbuf1, gbuf2,
              sg0, sg1, sg2, ss0, ss1, ss2):
    c = lax.axis_index("c")
    s = lax.axis_index("s")
    wid = s * _NC + c
    cbase = wid * nchunk_pw
    ebase = cbase * _K
    first, num = _node_chunks(s)
    ewb = (ewb0, ewb1, ewb2)
    idxb = (idxb0, idxb1, idxb2)
    colb = (colb0, colb1, colb2)
    gbuf = (gbuf0, gbuf1, gbuf2)
    sg = (sg0, sg1, sg2)
    ss = (ss0, ss1, ss2)

    pltpu.sync_copy(rows_hbm.at[pl.ds(ebase, nchunk_pw * _K)], rowball)

    @pl.loop(0, num)
    def _zero(k):
        pltpu.sync_copy(zeros, accum.at[pl.ds((first + k) * _RCH, _RCH)])

    plsc.subcore_barrier()

    def issue(q, p, toff):
        for g in range(_K // 16):
            sl = pl.ds(g * 16, 16)
            idxb[p][sl] = rowball[pl.ds(q * _K + g * 16, 16)] + toff
        pltpu.async_copy(cols_hbm.at[pl.ds(ebase + q * _K, _K)], colb[p], sg[p])
        pltpu.async_copy(ewc.at[pl.ds(ebase + q * _K, _K)],
                         ewb[p].at[pl.ds(16, _K)], sg[p])
        pltpu.async_copy(xp.at[idxb[p]], gbuf[p], sg[p])

    def process(q, p):
        pltpu.make_async_copy(cols_hbm.at[pl.ds(ebase + q * _K, _K)],
                              colb[p], sg[p]).wait()
        pltpu.make_async_copy(ewc.at[pl.ds(ebase + q * _K, _K)],
                              ewb[p].at[pl.ds(16, _K)], sg[p]).wait()
        pltpu.make_async_copy(xp.at[idxb[p]], gbuf[p], sg[p]).wait()
        for e in range(_K):
            spl = plsc.load_gather(ewb[p], [_full16(16 + e)])
            for j in range(_F // 16):
                cs = pl.ds(j * 16, 16)
                gbuf[p][e, cs] = gbuf[p][e, cs] * spl
        pltpu.async_copy(gbuf[p], accum.at[colb[p]], ss[p], add=True)

    def drain(p):
        pltpu.make_async_copy(gbuf[p], accum.at[colb[p]], ss[p]).wait()

    ntrip = (nchunk_pw - 2) // 3

    @pl.loop(0, _T)
    def _t_loop(t):
        toff = t * _N
        issue(0, 0, toff)
        issue(1, 1, toff)

        @pl.loop(0, ntrip)
        def _trip(j):
            i3 = 3 * j
            for r in range(3):
                i = i3 + r
                process(i, r)

                @pl.when(i >= 1)
                def _():
                    drain((r + 2) % 3)

                issue(i + 2, (r + 2) % 3, toff)

        process(nchunk_pw - 2, 0)
        drain(2)
        process(nchunk_pw - 1, 1)
        drain(0)
        drain(1)
        plsc.subcore_barrier()

        @pl.loop(0, num)
        def _dump(k):
            r0 = (first + k) * _RCH
            pltpu.sync_copy(accum.at[pl.ds(r0, _RCH)], gbuf0)
            pltpu.sync_copy(gbuf0, out.at[c, t, pl.ds(r0, _RCH)])

        plsc.subcore_barrier()


def _agg_sc(rows, cols, ewc, xp_flat, nchunk_pw):
    mesh = plsc.VectorSubcoreMesh(core_axis_name="c", subcore_axis_name="s")
    zeros = jnp.zeros((_RCH, _F), jnp.float32)
    ne_pw = nchunk_pw * _K
    return pl.kernel(
        functools.partial(_agg_body, nchunk_pw),
        out_type=jax.ShapeDtypeStruct((_NC, _T, _N, _F), jnp.float32),
        mesh=mesh,
        compiler_params=pltpu.CompilerParams(needs_layout_passes=False),
        scratch_types=[
            pltpu.VMEM_SHARED((_N, _F), jnp.float32),
            pltpu.VMEM((ne_pw,), jnp.int32),
            pltpu.VMEM((16 + _K,), jnp.float32),
            pltpu.VMEM((16 + _K,), jnp.float32),
            pltpu.VMEM((16 + _K,), jnp.float32),
            pltpu.VMEM((_K,), jnp.int32),
            pltpu.VMEM((_K,), jnp.int32),
            pltpu.VMEM((_K,), jnp.int32),
            pltpu.VMEM((_K,), jnp.int32),
            pltpu.VMEM((_K,), jnp.int32),
            pltpu.VMEM((_K,), jnp.int32),
            pltpu.VMEM((_K, _F), jnp.float32),
            pltpu.VMEM((_K, _F), jnp.float32),
            pltpu.VMEM((_K, _F), jnp.float32),
            pltpu.SemaphoreType.DMA,
            pltpu.SemaphoreType.DMA,
            pltpu.SemaphoreType.DMA,
            pltpu.SemaphoreType.DMA,
            pltpu.SemaphoreType.DMA,
            pltpu.SemaphoreType.DMA,
        ],
    )(rows, cols, ewc, xp_flat, zeros)


def _prep_body(x_ref, d_ref, xp_ref, dinv_ref, *, T):
    dsum = 1.0 + d_ref[0, :, 0] + d_ref[1, :, 0]
    dinv = lax.rsqrt(dsum)
    dinv_ref[...] = dinv[:, None]
    x = x_ref[...]
    xp_ref[...] = jnp.transpose(x, (1, 0, 2)) * dinv[None, :, None]


def _prep_tc(x_seq, degdump):
    N, T, F = x_seq.shape
    NB = 2000
    grid = (N // NB,)
    return pl.pallas_call(
        functools.partial(_prep_body, T=T),
        grid=grid,
        in_specs=[
            pl.BlockSpec((NB, T, F), lambda i: (i, 0, 0)),
            pl.BlockSpec((2, NB, _F), lambda i: (0, i, 0)),
        ],
        out_specs=[
            pl.BlockSpec((T, NB, F), lambda i: (0, i, 0)),
            pl.BlockSpec((NB, 1), lambda i: (i, 0)),
        ],
        out_shape=[
            jax.ShapeDtypeStruct((T, N, F), jnp.float32),
            jax.ShapeDtypeStruct((N, 1), jnp.float32),
        ],
    )(x_seq, degdump)


def _dense_body(S_ref, xp_ref, dinv_ref, Wz_ref, Wr_ref, Wh_ref,
                bz_ref, br_ref, bh_ref, Wlz_ref, Wlr_ref, Wlh_ref,
                blz_ref, blr_ref, blh_ref, Wout_ref, bout_ref, out_ref,
                *, T, NB, F, HS):
    f32 = jnp.float32
    dot = functools.partial(jnp.dot, preferred_element_type=f32)
    Wlz = Wlz_ref[...]
    Wlr = Wlr_ref[...]
    Wlh = Wlh_ref[...]
    Cz = dot(Wz_ref[...], Wlz[:HS])
    Cr = dot(Wr_ref[...], Wlr[:HS])
    Ch = dot(Wh_ref[...], Wlh[:HS])
    cz = dot(bz_ref[...], Wlz[:HS]) + blz_ref[...]
    cr = dot(br_ref[...], Wlr[:HS]) + blr_ref[...]
    ch = dot(bh_ref[...], Wlh[:HS]) + blh_ref[...]
    Uz, Ur, Uh = Wlz[HS:], Wlr[HS:], Wlh[HS:]

    dinv = dinv_ref[...]
    S = S_ref[...]
    xp = xp_ref[...]
    Ssum = S[0] + S[1]
    ys = [Ssum[0] + xp[0]]
    for t in range(1, T):
        ys.append(Ssum[t] - Ssum[t - 1] + xp[t])
    y = jnp.stack(ys, axis=0) * dinv[None, :, :]

    Ccat = jnp.concatenate([Cz, Cr, Ch], axis=1)
    P = dot(y.reshape(T * NB, F), Ccat).reshape(T, NB, 3 * HS)

    h = jnp.zeros((NB, HS), f32)
    for t in range(T):
        Z = jax.nn.sigmoid(P[t, :, :HS] + dot(h, Uz) + cz)
        R = jax.nn.sigmoid(P[t, :, HS:2 * HS] + dot(h, Ur) + cr)
        Ht = jnp.tanh(P[t, :, 2 * HS:] + dot(h * R, Uh) + ch)
        h = Z * h + (1.0 - Z) * Ht
    out_ref[...] = dot(h, Wout_ref[...]) + bout_ref[...]


def _dense_tc(S_cum, xp, dinv, Wz, Wr, Wh, bz, br, bh, Wlz, Wlr, Wlh,
              blz, blr, blh, Wout, bout):
    T, N, F = xp.shape
    HS = Wz.shape[1]
    NB = 1000
    grid = (N // NB,)
    full = lambda a: pl.BlockSpec(a.shape, lambda i: (0,) * a.ndim)
    return pl.pallas_call(
        functools.partial(_dense_body, T=T, NB=NB, F=F, HS=HS),
        grid=grid,
        in_specs=[
            pl.BlockSpec((2, T, NB, F), lambda i: (0, 0, i, 0)),
            pl.BlockSpec((T, NB, F), lambda i: (0, i, 0)),
            pl.BlockSpec((NB, 1), lambda i: (i, 0)),
            full(Wz), full(Wr), full(Wh),
            full(bz), full(br), full(bh),
            full(Wlz), full(Wlr), full(Wlh),
            full(blz), full(blr), full(blh),
            full(Wout), full(bout),
        ],
        out_specs=pl.BlockSpec((NB, 1), lambda i: (i, 0)),
        out_shape=jax.ShapeDtypeStruct((N, 1), jnp.float32),
    )(S_cum, xp, dinv, Wz, Wr, Wh, bz, br, bh, Wlz, Wlr, Wlh,
      blz, blr, blh, Wout, bout)


def kernel(x_seq, edge_index, edge_weight, Wz, bz, Wr, br, Wh, bh,
           Wlz, blz, Wlr, blr, Wlh, blh, Wout, bout):
    N, T, F = x_seq.shape
    HS = Wz.shape[1]
    E = edge_index.shape[1]
    nchunks = E // _K
    nchunk_pw = nchunks // _NW

    rows, cols = edge_index[0], edge_index[1]

    degdump = _deg_sc(cols, edge_weight, nchunk_pw)
    xp, dinv = _prep_tc(x_seq, degdump)
    S_cum = _agg_sc(rows, cols, edge_weight,
                    xp.reshape(T * N, F), nchunk_pw)

    return _dense_tc(
        S_cum, xp, dinv, Wz, Wr, Wh,
        bz.reshape(1, HS), br.reshape(1, HS), bh.reshape(1, HS),
        Wlz, Wlr, Wlh,
        blz.reshape(1, HS), blr.reshape(1, HS), blh.reshape(1, HS),
        Wout, bout.reshape(1, 1))
